# Initial kernel scaffold; baseline (speedup 1.0000x reference)
import jax
import jax.numpy as jnp
from jax.experimental import pallas as pl


def kernel(input, weight):
    raise NotImplementedError("write your pallas kernel here")


# speedup vs baseline: 4.1465x; 4.1465x over previous
import functools

import jax
import jax.numpy as jnp
from jax import lax
from jax.experimental import pallas as pl
from jax.experimental.pallas import tpu as pltpu
from jax.experimental.pallas import tpu_sc as plsc

_B_TOTAL = 16384 * 200
_D = 64
_NC, _NS = 2, 16
_NW = _NC * _NS
_B_PER_W = _B_TOTAL // _NW
_K = 8
_CHUNK = _K * 128
_G = _B_PER_W // _CHUNK
_ROWS_PER_W = _B_PER_W // 128


def _emb_body(idx_hbm, table_hbm, out_hbm, idx_v, rows_v, sem):
    sid = lax.axis_index("s")
    wid = sid * _NC + lax.axis_index("c")
    row_base = wid * _ROWS_PER_W

    def group(g, carry):
        r0 = row_base + g * _K
        pltpu.sync_copy(idx_hbm.at[pl.ds(r0, _K)], idx_v)
        copies = [
            pltpu.async_copy(table_hbm.at[idx_v.at[j]], rows_v.at[j], sem)
            for j in range(_K)
        ]
        for c in copies:
            c.wait()
        pltpu.sync_copy(rows_v, out_hbm.at[pl.ds(r0, _K)])
        return carry

    lax.fori_loop(0, _G, group, 0)


def kernel(input, weight):
    idx = input.reshape(_B_TOTAL // 128, 128).astype(jnp.int32)
    mesh = plsc.VectorSubcoreMesh(core_axis_name="c", subcore_axis_name="s")
    call = pl.kernel(
        _emb_body,
        out_type=jax.ShapeDtypeStruct((_B_TOTAL // 128, 128, _D), jnp.float32),
        mesh=mesh,
        scratch_types=[
            pltpu.VMEM((_K, 128), jnp.int32),
            pltpu.VMEM((_K, 128, _D), jnp.float32),
            pltpu.SemaphoreType.DMA,
        ],
        compiler_params=pltpu.CompilerParams(use_tc_tiling_on_sc=False),
    )
    out = call(idx, weight)
    return out.reshape(16384, 200, _D)


# speedup vs baseline: 5.5805x; 1.3458x over previous
import jax
import jax.numpy as jnp
from jax import lax
from jax.experimental import pallas as pl
from jax.experimental.pallas import tpu as pltpu
from jax.experimental.pallas import tpu_sc as plsc

_B_TOTAL = 16384 * 200
_D = 64
_V = 1000
_NC, _NS = 2, 16
_NW = _NC * _NS
_B_PER_W = _B_TOTAL // _NW
_K = 4
_G = _B_PER_W // (_K * 128)
_ROWS_PER_W = _B_PER_W // 128


def _emb_body(idx_hbm, table_hbm, out_hbm, table_sh, idx_v, rows_v, gsem, osem):
    sid = lax.axis_index("s")
    wid = sid * _NC + lax.axis_index("c")
    row_base = wid * _ROWS_PER_W

    pltpu.sync_copy(table_hbm, table_sh)

    def fire_group(g, b):
        r0 = row_base + g * _K
        pltpu.sync_copy(idx_hbm.at[pl.ds(r0, _K)], idx_v.at[b])
        for j in range(_K):
            pltpu.async_copy(table_sh.at[idx_v.at[b].at[j]], rows_v.at[b].at[j], gsem)

    def drain_group(b):
        for j in range(_K):
            pltpu.make_async_copy(
                table_sh.at[idx_v.at[b].at[j]], rows_v.at[b].at[j], gsem
            ).wait()

    def drain_out(b):
        pltpu.make_async_copy(
            rows_v.at[b], out_hbm.at[pl.ds(0, _K)], osem
        ).wait()

    fire_group(0, 0)

    def pair(p, carry):
        g0 = p * 2
        for b in range(2):
            gg = g0 + b
            nb = 1 - b
            drain_group(b)
            pltpu.async_copy(
                rows_v.at[b], out_hbm.at[pl.ds(row_base + gg * _K, _K)], osem
            )

            @pl.when(gg >= 1)
            def _():
                drain_out(nb)

            @pl.when(gg + 1 < _G)
            def _():
                fire_group(gg + 1, nb)
        return carry

    lax.fori_loop(0, _G // 2, pair, 0)
    drain_out(1)


def kernel(input, weight):
    idx = input.reshape(_B_TOTAL // 128, 128).astype(jnp.int32)
    mesh = plsc.VectorSubcoreMesh(core_axis_name="c", subcore_axis_name="s")
    call = pl.kernel(
        _emb_body,
        out_type=jax.ShapeDtypeStruct((_B_TOTAL // 128, 128, _D), jnp.float32),
        mesh=mesh,
        scratch_types=[
            pltpu.VMEM_SHARED((_V, _D), jnp.float32),
            pltpu.VMEM((2, _K, 128), jnp.int32),
            pltpu.VMEM((2, _K, 128, _D), jnp.float32),
            pltpu.SemaphoreType.DMA,
            pltpu.SemaphoreType.DMA,
        ],
        compiler_params=pltpu.CompilerParams(use_tc_tiling_on_sc=False),
    )
    out = call(idx, weight)
    return out.reshape(16384, 200, _D)
